# baseline (speedup 1.0000x reference)
import functools

import jax
import jax.numpy as jnp
from jax import lax
from jax.experimental import pallas as pl
from jax.experimental.pallas import tpu as pltpu
from jax.experimental.pallas import tpu_sc as plsc

B, L, D = 16, 576, 384
C = 1024
QK_DIM = 384
HEADS = 6
N = B * L
BLK = 1024

NC, NS, LN = 2, 16, 16
COLT_PER_CORE = 12
NCOLG = NC * COLT_PER_CORE
NCNT = NC * (NS - COLT_PER_CORE)
W = 128
NWIN = N // W
TCH = 2304
NTCH = N // TCH
WPC = TCH // W
CNT_WIN = NWIN // NCNT


def _assign_kernel(x_ref, means_ref, xn_ref, bkt_ref):
    x = x_ref[...]
    nrm = jnp.sqrt(jnp.sum(x * x, axis=-1, keepdims=True))
    xn = x / jnp.maximum(nrm, 1e-12)
    xn_ref[...] = xn
    dists = jax.lax.dot_general(
        xn, means_ref[...], (((1,), (1,)), ((), ())),
        preferred_element_type=jnp.float32)
    bkt_ref[...] = jnp.argmax(dists, axis=-1).astype(jnp.int32).reshape(1, 1, BLK)


def _sc_scatter(xn_hbm, bkt_hbm, zero_hbm,
                sums_out, counts_out, bkt_v, rows_v, acc):
    c = lax.axis_index("c")
    s = lax.axis_index("s")
    iota = jax.lax.iota(jnp.int32, LN)

    pltpu.sync_copy(zero_hbm, acc)
    pltpu.sync_copy(bkt_hbm, bkt_v)

    @pl.when(s < COLT_PER_CORE)
    def _col_tile():
        cg = c * COLT_PER_CORE + s
        for ch in range(NTCH):
            pltpu.sync_copy(
                xn_hbm.at[pl.ds(ch * TCH, TCH), pl.ds(cg * LN, LN)], rows_v)
            base_t = ch * TCH

            def body(t, carry):
                bj = plsc.load_gather(
                    bkt_v, [jnp.full((LN,), base_t + t, jnp.int32)])
                plsc.addupdate_scatter(acc, [bj, iota], rows_v[t, :])
                return carry

            lax.fori_loop(0, TCH, body, 0, unroll=8)
        pltpu.sync_copy(acc, sums_out.at[cg])

    @pl.when(s >= COLT_PER_CORE)
    def _cnt_tile():
        kc = c * (NS - COLT_PER_CORE) + (s - COLT_PER_CORE)
        tpc = N // NCNT
        ones_v = jnp.ones((LN,), jnp.float32)

        def cbody(t, carry):
            bj = plsc.load_gather(
                bkt_v, [jnp.full((LN,), kc * tpc + t, jnp.int32)])
            plsc.addupdate_scatter(acc, [bj, iota], ones_v)
            return carry

        lax.fori_loop(0, tpc, cbody, 0, unroll=8)
        pltpu.sync_copy(acc, counts_out.at[kc])


_sc_scatter_call = functools.partial(
    pl.kernel,
    out_type=[
        jax.ShapeDtypeStruct((NCOLG, C, LN), jnp.float32),
        jax.ShapeDtypeStruct((NCNT, C, LN), jnp.float32),
    ],
    mesh=plsc.VectorSubcoreMesh(core_axis_name="c", subcore_axis_name="s"),
    compiler_params=pltpu.CompilerParams(
        use_tc_tiling_on_sc=False, needs_layout_passes=False),
    scratch_types=[
        pltpu.VMEM((N,), jnp.int32),
        pltpu.VMEM((TCH, LN), jnp.float32),
        pltpu.VMEM((C, LN), jnp.float32),
    ],
)(_sc_scatter)


def _finalize_kernel(sums_ref, counts_ref, means_ref, wk_ref, wv_ref,
                     xg_ref, k_ref, v_ref):
    s = jnp.concatenate([sums_ref[g] for g in range(NCOLG)], axis=-1)
    sn = s / jnp.maximum(jnp.sqrt(jnp.sum(s * s, axis=-1, keepdims=True)), 1e-12)
    cnt = counts_ref[0, :, 0:1]
    for k in range(1, NCNT):
        cnt = cnt + counts_ref[k, :, 0:1]
    xg = jnp.where(cnt == 0.0, means_ref[...], sn)
    xg_ref[...] = xg
    k_ref[...] = jax.lax.dot_general(
        xg, wk_ref[...], (((1,), (1,)), ((), ())),
        preferred_element_type=jnp.float32)
    v_ref[...] = jax.lax.dot_general(
        xg, wv_ref[...], (((1,), (1,)), ((), ())),
        preferred_element_type=jnp.float32)


def kernel(normed_x, x_means, W_k, W_v):
    x = normed_x.reshape(N, D)
    mn = x_means / jnp.maximum(
        jnp.linalg.norm(x_means, axis=-1, keepdims=True), 1e-12)

    xn, bkt3 = pl.pallas_call(
        _assign_kernel,
        grid=(N // BLK,),
        in_specs=[
            pl.BlockSpec((BLK, D), lambda i: (i, 0)),
            pl.BlockSpec((C, D), lambda i: (0, 0)),
        ],
        out_specs=[
            pl.BlockSpec((BLK, D), lambda i: (i, 0)),
            pl.BlockSpec((1, 1, BLK), lambda i: (i, 0, 0)),
        ],
        out_shape=[
            jax.ShapeDtypeStruct((N, D), jnp.float32),
            jax.ShapeDtypeStruct((N // BLK, 1, BLK), jnp.int32),
        ],
    )(x, mn)

    bkt = bkt3.reshape(N)
    zero = jnp.zeros((C, LN), jnp.float32)
    sums24, counts8 = _sc_scatter_call(xn, bkt, zero)

    xg, k, v = pl.pallas_call(
        _finalize_kernel,
        out_shape=[
            jax.ShapeDtypeStruct((C, D), jnp.float32),
            jax.ShapeDtypeStruct((C, QK_DIM), jnp.float32),
            jax.ShapeDtypeStruct((C, D), jnp.float32),
        ],
    )(sums24, counts8, mn, W_k, W_v)

    k = k.reshape(C, HEADS, QK_DIM // HEADS).transpose(1, 0, 2)
    v = v.reshape(C, HEADS, D // HEADS).transpose(1, 0, 2)
    return (k, v, xg)


# speedup vs baseline: 1.3611x; 1.3611x over previous
import functools

import jax
import jax.numpy as jnp
from jax import lax
from jax.experimental import pallas as pl
from jax.experimental.pallas import tpu as pltpu
from jax.experimental.pallas import tpu_sc as plsc

B, L, D = 16, 576, 384
C = 1024
QK_DIM = 384
HEADS = 6
N = B * L
BLK = 1024

NC, NS, LN = 2, 16, 16
COLT_PER_CORE = 12
NCOLG = NC * COLT_PER_CORE
NCNT = NC * (NS - COLT_PER_CORE)
W = 128
NWIN = N // W
TCH = 2304
NTCH = N // TCH
WPC = TCH // W
CNT_WIN = NWIN // NCNT


def _assign_kernel(x_ref, means_ref, xn_ref, bkt_ref):
    x = x_ref[...]
    nrm = jnp.sqrt(jnp.sum(x * x, axis=-1, keepdims=True))
    xn = x / jnp.maximum(nrm, 1e-12)
    xn_ref[...] = xn
    dists = jax.lax.dot_general(
        xn, means_ref[...], (((1,), (1,)), ((), ())),
        preferred_element_type=jnp.float32)
    bkt_ref[...] = jnp.argmax(dists, axis=-1).astype(jnp.int32).reshape(1, 1, BLK)


_GDN = jax.lax.GatherDimensionNumbers(
    offset_dims=(), collapsed_slice_dims=(0,), start_index_map=(0,))


def _bcast_lane(v, j):
    idx = jnp.full((LN, 1), j, jnp.int32)
    return jax.lax.gather(
        v, idx, _GDN, (1,),
        mode=jax.lax.GatherScatterMode.PROMISE_IN_BOUNDS)


def _sc_scatter(xn_hbm, bkt_hbm, zero_hbm,
                sums_out, counts_out, bkt_v, rows_v, acc):
    c = lax.axis_index("c")
    s = lax.axis_index("s")
    iota = jax.lax.iota(jnp.int32, LN)

    pltpu.sync_copy(zero_hbm, acc)
    pltpu.sync_copy(bkt_hbm, bkt_v)

    @pl.when(s < COLT_PER_CORE)
    def _col_tile():
        cg = c * COLT_PER_CORE + s
        for ch in range(NTCH):
            pltpu.sync_copy(
                xn_hbm.at[pl.ds(ch * TCH, TCH), pl.ds(cg * LN, LN)], rows_v)
            base_t = ch * TCH

            def body(g, carry):
                bv = bkt_v[pl.ds(base_t + g * LN, LN)]
                rb = g * LN
                for j in range(LN):
                    bj = _bcast_lane(bv, j)
                    plsc.addupdate_scatter(acc, [bj, iota], rows_v[rb + j, :])
                return carry

            lax.fori_loop(0, TCH // LN, body, 0)
        pltpu.sync_copy(acc, sums_out.at[pl.ds(0, C), pl.ds(cg * LN, LN)])

    @pl.when(s >= COLT_PER_CORE)
    def _cnt_tile():
        kc = c * (NS - COLT_PER_CORE) + (s - COLT_PER_CORE)
        tpc = N // NCNT
        ones_v = jnp.ones((LN,), jnp.float32)

        def cbody(g, carry):
            bv = bkt_v[pl.ds(kc * tpc + g * LN, LN)]
            for j in range(LN):
                bj = _bcast_lane(bv, j)
                plsc.addupdate_scatter(acc, [bj, iota], ones_v)
            return carry

        lax.fori_loop(0, tpc // LN, cbody, 0)
        pltpu.sync_copy(acc, counts_out.at[kc])


_sc_scatter_call = functools.partial(
    pl.kernel,
    out_type=[
        jax.ShapeDtypeStruct((C, D), jnp.float32),
        jax.ShapeDtypeStruct((NCNT, C, LN), jnp.float32),
    ],
    mesh=plsc.VectorSubcoreMesh(core_axis_name="c", subcore_axis_name="s"),
    compiler_params=pltpu.CompilerParams(
        use_tc_tiling_on_sc=False, needs_layout_passes=False),
    scratch_types=[
        pltpu.VMEM((N,), jnp.int32),
        pltpu.VMEM((TCH, LN), jnp.float32),
        pltpu.VMEM((C, LN), jnp.float32),
    ],
)(_sc_scatter)


def _finalize_kernel(sums_ref, counts_ref, means_ref, wk_ref, wv_ref,
                     xg_ref, k_ref, v_ref):
    s = sums_ref[...]
    sn = s / jnp.maximum(jnp.sqrt(jnp.sum(s * s, axis=-1, keepdims=True)), 1e-12)
    cnt = counts_ref[0, :, 0:1]
    for k in range(1, NCNT):
        cnt = cnt + counts_ref[k, :, 0:1]
    xg = jnp.where(cnt == 0.0, means_ref[...], sn)
    xg_ref[...] = xg
    k_ref[...] = jax.lax.dot_general(
        xg, wk_ref[...], (((1,), (1,)), ((), ())),
        preferred_element_type=jnp.float32)
    v_ref[...] = jax.lax.dot_general(
        xg, wv_ref[...], (((1,), (1,)), ((), ())),
        preferred_element_type=jnp.float32)


def kernel(normed_x, x_means, W_k, W_v):
    x = normed_x.reshape(N, D)
    mn = x_means / jnp.maximum(
        jnp.linalg.norm(x_means, axis=-1, keepdims=True), 1e-12)

    xn, bkt3 = pl.pallas_call(
        _assign_kernel,
        grid=(N // BLK,),
        in_specs=[
            pl.BlockSpec((BLK, D), lambda i: (i, 0)),
            pl.BlockSpec((C, D), lambda i: (0, 0)),
        ],
        out_specs=[
            pl.BlockSpec((BLK, D), lambda i: (i, 0)),
            pl.BlockSpec((1, 1, BLK), lambda i: (i, 0, 0)),
        ],
        out_shape=[
            jax.ShapeDtypeStruct((N, D), jnp.float32),
            jax.ShapeDtypeStruct((N // BLK, 1, BLK), jnp.int32),
        ],
    )(x, mn)

    bkt = bkt3.reshape(N)
    zero = jnp.zeros((C, LN), jnp.float32)
    sums, counts8 = _sc_scatter_call(xn, bkt, zero)

    xg, k, v = pl.pallas_call(
        _finalize_kernel,
        out_shape=[
            jax.ShapeDtypeStruct((C, D), jnp.float32),
            jax.ShapeDtypeStruct((C, QK_DIM), jnp.float32),
            jax.ShapeDtypeStruct((C, D), jnp.float32),
        ],
    )(sums, counts8, mn, W_k, W_v)

    k = k.reshape(C, HEADS, QK_DIM // HEADS).transpose(1, 0, 2)
    v = v.reshape(C, HEADS, D // HEADS).transpose(1, 0, 2)
    return (k, v, xg)
